# baseline (speedup 1.0000x reference)
import functools

import jax
import jax.numpy as jnp
from jax import lax
from jax.experimental import pallas as pl
from jax.experimental.pallas import tpu as pltpu
from jax.experimental.pallas import tpu_sc as plsc

_MAX_NORM = 7.0
_EPS = 1e-7

_NC = 2
_NS = 16
_NW = _NC * _NS
_L = 16

_ROWS_PER_CHUNK = 16


def _newton_rsqrt(s):
    y = plsc.bitcast(s, jnp.int32)
    y = jnp.int32(0x5F3759DF) - (y >> 1)
    x = plsc.bitcast(y, jnp.float32)
    for _ in range(3):
        x = x * (1.5 - 0.5 * s * x * x)
    return x


@functools.partial(jax.jit, static_argnums=(2, 3, 4))
def _sc_lookup(table, node_id, b, h, d):
    rows_per_w = b // _NW
    n_chunks = rows_per_w // _ROWS_PER_CHUNK
    idx_per_chunk = _ROWS_PER_CHUNK * h
    groups = idx_per_chunk // _L

    mesh = plsc.VectorSubcoreMesh(core_axis_name="c", subcore_axis_name="s")

    @functools.partial(
        pl.kernel,
        out_type=jax.ShapeDtypeStruct((b, h, d), jnp.float32),
        mesh=mesh,
        scratch_types=[
            pltpu.VMEM((_ROWS_PER_CHUNK, h), jnp.int32),
            pltpu.VMEM((_ROWS_PER_CHUNK, h), jnp.int32),
            pltpu.VMEM((_ROWS_PER_CHUNK, h, d), jnp.float32),
            pltpu.VMEM((_ROWS_PER_CHUNK, h, d), jnp.float32),
            pltpu.SemaphoreType.DMA,
            pltpu.SemaphoreType.DMA,
            pltpu.SemaphoreType.DMA,
            pltpu.SemaphoreType.DMA,
        ],
        compiler_params=pltpu.CompilerParams(needs_layout_passes=False,
                                             use_tc_tiling_on_sc=False),
    )
    def k(table_hbm, idx_hbm, out_hbm, idx0, idx1, rows0, rows1,
          gsem0, gsem1, wsem0, wsem1):
        wid = lax.axis_index("s") * _NC + lax.axis_index("c")
        w_row0 = wid * rows_per_w
        lane = lax.iota(jnp.int32, _L)
        idx_bufs = (idx0, idx1)
        row_bufs = (rows0, rows1)
        gsems = (gsem0, gsem1)
        wsems = (wsem0, wsem1)

        def load_idx(c, buf):
            r0 = pl.multiple_of(w_row0 + c * _ROWS_PER_CHUNK, 8)
            pltpu.sync_copy(idx_hbm.at[pl.ds(r0, _ROWS_PER_CHUNK)], buf)

        def fire_gathers(bi):
            for r in range(_ROWS_PER_CHUNK):
                pltpu.async_copy(table_hbm.at[idx_bufs[bi].at[r]],
                                 row_bufs[bi].at[r], gsems[bi])

        def drain_gathers(bi):
            for r in range(_ROWS_PER_CHUNK):
                pltpu.make_async_copy(table_hbm.at[idx_bufs[bi].at[r]],
                                      row_bufs[bi].at[r], gsems[bi]).wait()

        def compute(bi):
            rows_v = row_bufs[bi]
            zero = jnp.zeros((_L,), jnp.int32)

            @plsc.parallel_loop(0, groups, 1, unroll=2, carry=jnp.int32(0))
            def _group(g, j):
                base = (g * _L + lane) * d
                vals = []
                ssq = jnp.zeros((_L,), jnp.float32)
                for col in range(d):
                    v = plsc.load_gather(rows_v, [zero, zero, base + col])
                    vals.append(v)
                    ssq = ssq + v * v
                norm = ssq * _newton_rsqrt(ssq)
                scale = jnp.minimum(1.0, _MAX_NORM / (norm + _EPS))
                for col in range(d):
                    plsc.store_scatter(rows_v, [zero, zero, base + col],
                                       vals[col] * scale)
                return j

        def fire_writeback(c, bi):
            o0 = pl.multiple_of(w_row0 + c * _ROWS_PER_CHUNK, 8)
            return pltpu.async_copy(row_bufs[bi],
                                    out_hbm.at[pl.ds(o0, _ROWS_PER_CHUNK)],
                                    wsems[bi])

        def wait_writeback(c, bi):
            o0 = pl.multiple_of(w_row0 + c * _ROWS_PER_CHUNK, 8)
            pltpu.make_async_copy(row_bufs[bi],
                                  out_hbm.at[pl.ds(o0, _ROWS_PER_CHUNK)],
                                  wsems[bi]).wait()

        load_idx(0, idx0)
        fire_gathers(0)
        load_idx(1, idx1)
        fire_gathers(1)

        def outer_body(o, _):
            for bi in range(2):
                c = 2 * o + bi
                drain_gathers(bi)
                compute(bi)
                fire_writeback(c, bi)

                @pl.when(o < n_chunks // 2 - 1)
                def _prefetch():
                    load_idx(c + 2, idx_bufs[bi])
                    wait_writeback(c, bi)
                    fire_gathers(bi)

            return 0

        lax.fori_loop(0, n_chunks // 2, outer_body, 0)
        wait_writeback(n_chunks - 2, 0)
        wait_writeback(n_chunks - 1, 1)

    return k(table, node_id)


def kernel(node_id, table):
    b, h = node_id.shape
    d = table.shape[1]
    return _sc_lookup(table, node_id, b, h, d)


# speedup vs baseline: 1.0856x; 1.0856x over previous
import functools

import jax
import jax.numpy as jnp
from jax import lax
from jax.experimental import pallas as pl
from jax.experimental.pallas import tpu as pltpu
from jax.experimental.pallas import tpu_sc as plsc

_MAX_NORM = 7.0
_EPS = 1e-7

_NC = 2
_NS = 16
_NW = _NC * _NS
_L = 16

_ROWS_PER_CHUNK = 16


def _newton_rsqrt(s):
    y = plsc.bitcast(s, jnp.int32)
    y = jnp.int32(0x5F3759DF) - (y >> 1)
    x = plsc.bitcast(y, jnp.float32)
    for _ in range(3):
        x = x * (1.5 - 0.5 * s * x * x)
    return x


@functools.partial(jax.jit, static_argnums=(2, 3, 4))
def _sc_lookup(table, node_id, b, h, d):
    rows_per_w = b // _NW
    n_chunks = rows_per_w // _ROWS_PER_CHUNK
    idx_per_chunk = _ROWS_PER_CHUNK * h
    groups = idx_per_chunk // _L

    mesh = plsc.VectorSubcoreMesh(core_axis_name="c", subcore_axis_name="s")

    @functools.partial(
        pl.kernel,
        out_type=jax.ShapeDtypeStruct((b, h, d), jnp.float32),
        mesh=mesh,
        scratch_types=[
            pltpu.VMEM((_ROWS_PER_CHUNK, h), jnp.int32),
            pltpu.VMEM((_ROWS_PER_CHUNK, h), jnp.int32),
            pltpu.VMEM((_ROWS_PER_CHUNK, h, d), jnp.float32),
            pltpu.VMEM((_ROWS_PER_CHUNK, h, d), jnp.float32),
            pltpu.SemaphoreType.DMA,
            pltpu.SemaphoreType.DMA,
            pltpu.SemaphoreType.DMA,
            pltpu.SemaphoreType.DMA,
        ],
        compiler_params=pltpu.CompilerParams(needs_layout_passes=False,
                                             use_tc_tiling_on_sc=False),
    )
    def k(table_hbm, idx_hbm, out_hbm, idx0, idx1, rows0, rows1,
          gsem0, gsem1, wsem0, wsem1):
        wid = lax.axis_index("s") * _NC + lax.axis_index("c")
        w_row0 = wid * rows_per_w
        lane = lax.iota(jnp.int32, _L)
        idx_bufs = (idx0, idx1)
        row_bufs = (rows0, rows1)
        gsems = (gsem0, gsem1)
        wsems = (wsem0, wsem1)

        def load_idx(c, buf):
            r0 = pl.multiple_of(w_row0 + c * _ROWS_PER_CHUNK, 8)
            pltpu.sync_copy(idx_hbm.at[pl.ds(r0, _ROWS_PER_CHUNK)], buf)

        def fire_gathers(bi):
            for r in range(_ROWS_PER_CHUNK):
                pltpu.async_copy(table_hbm.at[idx_bufs[bi].at[r]],
                                 row_bufs[bi].at[r], gsems[bi])

        def drain_gathers(bi):
            for r in range(_ROWS_PER_CHUNK):
                pltpu.make_async_copy(table_hbm.at[idx_bufs[bi].at[r]],
                                      row_bufs[bi].at[r], gsems[bi]).wait()

        def compute(bi):
            rows_v = row_bufs[bi]
            zero = jnp.zeros((_L,), jnp.int32)

            @plsc.parallel_loop(0, groups, 1, unroll=1, carry=jnp.int32(0))
            def _group(g, j):
                base = (g * _L + lane) * d
                vals = []
                ssq = jnp.zeros((_L,), jnp.float32)
                for col in range(d):
                    v = plsc.load_gather(rows_v, [zero, zero, base + col])
                    vals.append(v)
                    ssq = ssq + v * v
                norm = ssq * _newton_rsqrt(ssq)
                scale = jnp.minimum(1.0, _MAX_NORM / (norm + _EPS))
                for col in range(d):
                    plsc.store_scatter(rows_v, [zero, zero, base + col],
                                       vals[col] * scale)
                return j

        def fire_writeback(c, bi):
            o0 = pl.multiple_of(w_row0 + c * _ROWS_PER_CHUNK, 8)
            return pltpu.async_copy(row_bufs[bi],
                                    out_hbm.at[pl.ds(o0, _ROWS_PER_CHUNK)],
                                    wsems[bi])

        def wait_writeback(c, bi):
            o0 = pl.multiple_of(w_row0 + c * _ROWS_PER_CHUNK, 8)
            pltpu.make_async_copy(row_bufs[bi],
                                  out_hbm.at[pl.ds(o0, _ROWS_PER_CHUNK)],
                                  wsems[bi]).wait()

        load_idx(0, idx0)
        fire_gathers(0)
        load_idx(1, idx1)
        fire_gathers(1)

        def outer_body(o, _):
            for bi in range(2):
                c = 2 * o + bi
                drain_gathers(bi)
                compute(bi)
                fire_writeback(c, bi)

                @pl.when(o < n_chunks // 2 - 1)
                def _prefetch():
                    load_idx(c + 2, idx_bufs[bi])
                    wait_writeback(c, bi)
                    fire_gathers(bi)

            return 0

        lax.fori_loop(0, n_chunks // 2, outer_body, 0)
        wait_writeback(n_chunks - 2, 0)
        wait_writeback(n_chunks - 1, 1)

    return k(table, node_id)


def kernel(node_id, table):
    b, h = node_id.shape
    d = table.shape[1]
    return _sc_lookup(table, node_id, b, h, d)


# speedup vs baseline: 1.4419x; 1.3283x over previous
import functools

import jax
import jax.numpy as jnp
from jax import lax
from jax.experimental import pallas as pl
from jax.experimental.pallas import tpu as pltpu
from jax.experimental.pallas import tpu_sc as plsc

_MAX_NORM = 7.0
_EPS = 1e-7

_NC = 2
_NS = 16
_NW = _NC * _NS
_L = 16

_ROWS_PER_CHUNK = 16


def _newton_rsqrt(s):
    y = plsc.bitcast(s, jnp.int32)
    y = jnp.int32(0x5F3759DF) - (y >> 1)
    x = plsc.bitcast(y, jnp.float32)
    for _ in range(3):
        x = x * (1.5 - 0.5 * s * x * x)
    return x


@functools.partial(jax.jit, static_argnums=(2, 3, 4))
def _sc_lookup(table, node_id, b, h, d):
    assert d & (d - 1) == 0, "column-skew addressing needs power-of-two d"
    rows_per_w = b // _NW
    n_chunks = rows_per_w // _ROWS_PER_CHUNK
    idx_per_chunk = _ROWS_PER_CHUNK * h
    groups = idx_per_chunk // _L

    mesh = plsc.VectorSubcoreMesh(core_axis_name="c", subcore_axis_name="s")

    @functools.partial(
        pl.kernel,
        out_type=jax.ShapeDtypeStruct((b, h, d), jnp.float32),
        mesh=mesh,
        scratch_types=[
            pltpu.VMEM((_ROWS_PER_CHUNK, h), jnp.int32),
            pltpu.VMEM((_ROWS_PER_CHUNK, h), jnp.int32),
            pltpu.VMEM((_ROWS_PER_CHUNK, h, d), jnp.float32),
            pltpu.VMEM((_ROWS_PER_CHUNK, h, d), jnp.float32),
            pltpu.SemaphoreType.DMA,
            pltpu.SemaphoreType.DMA,
            pltpu.SemaphoreType.DMA,
            pltpu.SemaphoreType.DMA,
        ],
        compiler_params=pltpu.CompilerParams(needs_layout_passes=False,
                                             use_tc_tiling_on_sc=False),
    )
    def k(table_hbm, idx_hbm, out_hbm, idx0, idx1, rows0, rows1,
          gsem0, gsem1, wsem0, wsem1):
        wid = lax.axis_index("s") * _NC + lax.axis_index("c")
        w_row0 = wid * rows_per_w
        lane = lax.iota(jnp.int32, _L)
        idx_bufs = (idx0, idx1)
        row_bufs = (rows0, rows1)
        gsems = (gsem0, gsem1)
        wsems = (wsem0, wsem1)

        def load_idx(c, buf):
            r0 = pl.multiple_of(w_row0 + c * _ROWS_PER_CHUNK, 8)
            pltpu.sync_copy(idx_hbm.at[pl.ds(r0, _ROWS_PER_CHUNK)], buf)

        def fire_gathers(bi):
            for r in range(_ROWS_PER_CHUNK):
                pltpu.async_copy(table_hbm.at[idx_bufs[bi].at[r]],
                                 row_bufs[bi].at[r], gsems[bi])

        def drain_gathers(bi):
            for r in range(_ROWS_PER_CHUNK):
                pltpu.make_async_copy(table_hbm.at[idx_bufs[bi].at[r]],
                                      row_bufs[bi].at[r], gsems[bi]).wait()

        def compute(bi):
            rows_v = row_bufs[bi]
            zero = jnp.zeros((_L,), jnp.int32)

            @plsc.parallel_loop(0, groups, 1, unroll=1, carry=jnp.int32(0))
            def _group(g, j):
                base = (g * _L + lane) * d
                vals = []
                ssq = jnp.zeros((_L,), jnp.float32)
                for col in range(d):
                    a = base + ((col + lane) & (d - 1))
                    v = plsc.load_gather(rows_v, [zero, zero, a])
                    vals.append(v)
                    ssq = ssq + v * v
                norm = ssq * _newton_rsqrt(ssq)
                scale = jnp.minimum(1.0, _MAX_NORM / (norm + _EPS))
                for col in range(d):
                    a = base + ((col + lane) & (d - 1))
                    plsc.store_scatter(rows_v, [zero, zero, a],
                                       vals[col] * scale)
                return j

        def fire_writeback(c, bi):
            o0 = pl.multiple_of(w_row0 + c * _ROWS_PER_CHUNK, 8)
            return pltpu.async_copy(row_bufs[bi],
                                    out_hbm.at[pl.ds(o0, _ROWS_PER_CHUNK)],
                                    wsems[bi])

        def wait_writeback(c, bi):
            o0 = pl.multiple_of(w_row0 + c * _ROWS_PER_CHUNK, 8)
            pltpu.make_async_copy(row_bufs[bi],
                                  out_hbm.at[pl.ds(o0, _ROWS_PER_CHUNK)],
                                  wsems[bi]).wait()

        load_idx(0, idx0)
        fire_gathers(0)
        load_idx(1, idx1)
        fire_gathers(1)

        def outer_body(o, _):
            for bi in range(2):
                c = 2 * o + bi
                drain_gathers(bi)
                compute(bi)
                fire_writeback(c, bi)

                @pl.when(o < n_chunks // 2 - 1)
                def _prefetch():
                    load_idx(c + 2, idx_bufs[bi])
                    wait_writeback(c, bi)
                    fire_gathers(bi)

            return 0

        lax.fori_loop(0, n_chunks // 2, outer_body, 0)
        wait_writeback(n_chunks - 2, 0)
        wait_writeback(n_chunks - 1, 1)

    return k(table, node_id)


def kernel(node_id, table):
    b, h = node_id.shape
    d = table.shape[1]
    return _sc_lookup(table, node_id, b, h, d)


# speedup vs baseline: 1.5458x; 1.0720x over previous
import functools

import jax
import jax.numpy as jnp
from jax import lax
from jax.experimental import pallas as pl
from jax.experimental.pallas import tpu as pltpu
from jax.experimental.pallas import tpu_sc as plsc

_MAX_NORM = 7.0
_EPS = 1e-7

_NC = 2
_NS = 16
_NW = _NC * _NS
_L = 16

_ROWS_PER_CHUNK = 16


def _newton_rsqrt(s):
    y = plsc.bitcast(s, jnp.int32)
    y = jnp.int32(0x5F3759DF) - (y >> 1)
    x = plsc.bitcast(y, jnp.float32)
    for _ in range(3):
        x = x * (1.5 - 0.5 * s * x * x)
    return x


@functools.partial(jax.jit, static_argnums=(2, 3, 4))
def _sc_lookup(table, node_id, b, h, d):
    assert d & (d - 1) == 0, "column-skew addressing needs power-of-two d"
    rows_per_w = b // _NW
    n_chunks = rows_per_w // _ROWS_PER_CHUNK
    idx_per_chunk = _ROWS_PER_CHUNK * h
    groups = idx_per_chunk // _L

    mesh = plsc.VectorSubcoreMesh(core_axis_name="c", subcore_axis_name="s")

    @functools.partial(
        pl.kernel,
        out_type=jax.ShapeDtypeStruct((b, h, d), jnp.float32),
        mesh=mesh,
        scratch_types=[
            pltpu.VMEM((_ROWS_PER_CHUNK, h), jnp.int32),
            pltpu.VMEM((_ROWS_PER_CHUNK, h), jnp.int32),
            pltpu.VMEM((_ROWS_PER_CHUNK, h, d), jnp.float32),
            pltpu.VMEM((_ROWS_PER_CHUNK, h, d), jnp.float32),
            pltpu.SemaphoreType.DMA,
            pltpu.SemaphoreType.DMA,
            pltpu.SemaphoreType.DMA,
            pltpu.SemaphoreType.DMA,
        ],
        compiler_params=pltpu.CompilerParams(needs_layout_passes=False,
                                             use_tc_tiling_on_sc=False),
    )
    def k(table_hbm, idx_hbm, out_hbm, idx0, idx1, rows0, rows1,
          gsem0, gsem1, wsem0, wsem1):
        wid = lax.axis_index("s") * _NC + lax.axis_index("c")
        w_row0 = wid * rows_per_w
        lane = lax.iota(jnp.int32, _L)
        idx_bufs = (idx0, idx1)
        row_bufs = (rows0, rows1)
        gsems = (gsem0, gsem1)
        wsems = (wsem0, wsem1)

        def load_idx(c, buf):
            r0 = pl.multiple_of(w_row0 + c * _ROWS_PER_CHUNK, 8)
            pltpu.sync_copy(idx_hbm.at[pl.ds(r0, _ROWS_PER_CHUNK)], buf)

        def fire_gathers(bi):
            for r in range(_ROWS_PER_CHUNK):
                pltpu.async_copy(table_hbm.at[idx_bufs[bi].at[r]],
                                 row_bufs[bi].at[r], gsems[bi])

        def drain_gathers(bi):
            for r in range(_ROWS_PER_CHUNK):
                pltpu.make_async_copy(table_hbm.at[idx_bufs[bi].at[r]],
                                      row_bufs[bi].at[r], gsems[bi]).wait()

        def compute(bi):
            rows_v = row_bufs[bi]
            zero = jnp.zeros((_L,), jnp.int32)

            @plsc.parallel_loop(0, groups, 1, unroll=1, carry=jnp.int32(0))
            def _group(g, j):
                base = (g * _L + lane) * d
                vals = []
                ssq = jnp.zeros((_L,), jnp.float32)
                for col in range(d):
                    a = base + ((col + lane) & (d - 1))
                    v = plsc.load_gather(rows_v, [zero, zero, a])
                    vals.append(v)
                    ssq = ssq + v * v
                norm = ssq * _newton_rsqrt(ssq)

                @pl.when(lax.reduce_max(norm, (0,)) > _MAX_NORM)
                def _rescale():
                    scale = jnp.minimum(1.0,
                                        _MAX_NORM / (norm + _EPS))
                    for col in range(d):
                        a = base + ((col + lane) & (d - 1))
                        plsc.store_scatter(rows_v, [zero, zero, a],
                                           vals[col] * scale)

                return j

        def fire_writeback(c, bi):
            o0 = pl.multiple_of(w_row0 + c * _ROWS_PER_CHUNK, 8)
            return pltpu.async_copy(row_bufs[bi],
                                    out_hbm.at[pl.ds(o0, _ROWS_PER_CHUNK)],
                                    wsems[bi])

        def wait_writeback(c, bi):
            o0 = pl.multiple_of(w_row0 + c * _ROWS_PER_CHUNK, 8)
            pltpu.make_async_copy(row_bufs[bi],
                                  out_hbm.at[pl.ds(o0, _ROWS_PER_CHUNK)],
                                  wsems[bi]).wait()

        load_idx(0, idx0)
        fire_gathers(0)
        load_idx(1, idx1)
        fire_gathers(1)

        def outer_body(o, _):
            for bi in range(2):
                c = 2 * o + bi
                drain_gathers(bi)
                compute(bi)
                fire_writeback(c, bi)

                @pl.when(o < n_chunks // 2 - 1)
                def _prefetch():
                    load_idx(c + 2, idx_bufs[bi])
                    wait_writeback(c, bi)
                    fire_gathers(bi)

            return 0

        lax.fori_loop(0, n_chunks // 2, outer_body, 0)
        wait_writeback(n_chunks - 2, 0)
        wait_writeback(n_chunks - 1, 1)

    return k(table, node_id)


def kernel(node_id, table):
    b, h = node_id.shape
    d = table.shape[1]
    return _sc_lookup(table, node_id, b, h, d)
